# Initial kernel scaffold; baseline (speedup 1.0000x reference)
import jax
import jax.numpy as jnp
from jax.experimental import pallas as pl


def kernel(n_range, s, idx):
    raise NotImplementedError("write your pallas kernel here")


# speedup vs baseline: 3.7544x; 3.7544x over previous
import functools

import jax
import jax.numpy as jnp
from jax import lax
from jax.experimental import pallas as pl
from jax.experimental.pallas import tpu as pltpu
from jax.experimental.pallas import tpu_sc as plsc

N = 1_000_000
NIDX = 65536
NW = 32
CHUNK = 31264
LAST = N - 31 * CHUNK
L = 16


@functools.partial(
    pl.kernel,
    mesh=plsc.VectorSubcoreMesh(core_axis_name="c", subcore_axis_name="s"),
    out_type=jax.ShapeDtypeStruct((N,), jnp.float32),
    scratch_types=[
        pltpu.VMEM((NIDX,), jnp.int32),
        pltpu.VMEM((CHUNK,), jnp.float32),
        pltpu.SemaphoreType.DMA,
    ],
    compiler_params=pltpu.CompilerParams(needs_layout_passes=False),
)
def _scatter_ones(idx_hbm, out_hbm, idx_v, chunk_v, sem):
    wid = lax.axis_index("s") * 2 + lax.axis_index("c")
    base = wid * CHUNK

    cp = pltpu.async_copy(idx_hbm, idx_v, sem)

    zeros = jnp.zeros((L,), jnp.float32)

    def zero_body(i, carry):
        chunk_v[pl.ds(i * L, L)] = zeros
        return carry

    lax.fori_loop(0, CHUNK // L, zero_body, 0)
    cp.wait()

    ones = jnp.ones((L,), jnp.float32)

    def scan_body(j, carry):
        t = idx_v[pl.ds(j * L, L)]
        m = (t >= base) & (t < base + CHUNK)
        loc = jnp.where(m, t - base, 0)
        plsc.store_scatter(chunk_v, [loc], ones, mask=m)
        return carry

    lax.fori_loop(0, NIDX // L, scan_body, 0)

    @pl.when(wid < NW - 1)
    def _():
        pltpu.sync_copy(chunk_v.at[pl.ds(0, CHUNK)], out_hbm.at[pl.ds(base, CHUNK)])

    @pl.when(wid == NW - 1)
    def _():
        pltpu.sync_copy(chunk_v.at[pl.ds(0, LAST)], out_hbm.at[pl.ds(base, LAST)])


def kernel(n_range, s, idx):
    del n_range, s
    return (_scatter_ones(idx.astype(jnp.int32)),)


# speedup vs baseline: 7.2630x; 1.9345x over previous
import functools

import jax
import jax.numpy as jnp
from jax import lax
from jax.experimental import pallas as pl
from jax.experimental.pallas import tpu as pltpu
from jax.experimental.pallas import tpu_sc as plsc

N = 1_000_000
NIDX = 65536
NW = 32
CHUNK = 31360
LAST = N - 31 * CHUNK
L = 16


@functools.partial(
    pl.kernel,
    mesh=plsc.VectorSubcoreMesh(core_axis_name="c", subcore_axis_name="s"),
    out_type=jax.ShapeDtypeStruct((N,), jnp.float32),
    scratch_types=[
        pltpu.VMEM((NIDX,), jnp.int32),
        pltpu.VMEM((CHUNK,), jnp.float32),
        pltpu.SemaphoreType.DMA,
    ],
    compiler_params=pltpu.CompilerParams(needs_layout_passes=False),
)
def _scatter_ones(idx_hbm, out_hbm, idx_v, chunk_v, sem):
    wid = lax.axis_index("s") * 2 + lax.axis_index("c")
    base = wid * CHUNK

    cp = pltpu.async_copy(idx_hbm, idx_v, sem)

    zeros = jnp.zeros((L,), jnp.float32)

    @plsc.parallel_loop(0, CHUNK // L, unroll=8)
    def _zero_body(i):
        chunk_v[pl.ds(i * L, L)] = zeros

    cp.wait()

    ones = jnp.ones((L,), jnp.float32)
    hi = base + CHUNK

    @plsc.parallel_loop(0, NIDX // L, unroll=8)
    def _scan_body(j):
        t = idx_v[pl.ds(j * L, L)]
        m = (t >= base) & (t < hi)
        loc = jnp.where(m, t - base, 0)
        plsc.store_scatter(chunk_v, [loc], ones, mask=m)

    @pl.when(wid < NW - 1)
    def _():
        pltpu.sync_copy(chunk_v.at[pl.ds(0, CHUNK)], out_hbm.at[pl.ds(base, CHUNK)])

    @pl.when(wid == NW - 1)
    def _():
        pltpu.sync_copy(chunk_v.at[pl.ds(0, LAST)], out_hbm.at[pl.ds(base, LAST)])


def kernel(n_range, s, idx):
    del n_range, s
    return (_scatter_ones(idx.astype(jnp.int32)),)
